# Initial kernel scaffold; baseline (speedup 1.0000x reference)
import functools

import jax
import jax.numpy as jnp
from jax import lax
from jax.experimental import pallas as pl
from jax.experimental.pallas import tpu as pltpu
from jax.experimental.pallas import tpu_sc as plsc

N_NODES = 10000
NNZ = 320000
D_FEAT = 128

NUM_CORES = 2
NUM_SUBCORES = 16
NW = NUM_CORES * NUM_SUBCORES
PER_W = NNZ // NW
CHUNK = 80
N_CHUNKS = PER_W // CHUNK


@functools.partial(
    pl.kernel,
    out_type=jax.ShapeDtypeStruct((NNZ, D_FEAT), jnp.float32),
    mesh=plsc.VectorSubcoreMesh(core_axis_name="c", subcore_axis_name="s"),
    scratch_types=[
        pltpu.VMEM((N_CHUNKS, CHUNK), jnp.int32),
        pltpu.VMEM((CHUNK, D_FEAT), jnp.float32),
        pltpu.SemaphoreType.DMA,
    ],
)
def _unpool(x_hbm, idx_hbm, out_hbm, idx_v, rows_v, sem):
    wid = lax.axis_index("s") * NUM_CORES + lax.axis_index("c")
    base = wid * PER_W
    pltpu.sync_copy(idx_hbm.at[wid], idx_v)

    def body(c, _):
        pltpu.async_copy(x_hbm.at[idx_v.at[c]], rows_v, sem).wait()
        pltpu.sync_copy(rows_v, out_hbm.at[pl.ds(base + c * CHUNK, CHUNK)])
        return ()

    lax.fori_loop(0, N_CHUNKS, body, (), unroll=False)


def kernel(X, tarX_indices):
    idx = tarX_indices[0].astype(jnp.int32).reshape(NW, N_CHUNKS, CHUNK)
    return _unpool(X, idx)


# speedup vs baseline: 1.6269x; 1.6269x over previous
import functools

import jax
import jax.numpy as jnp
from jax import lax
from jax.experimental import pallas as pl
from jax.experimental.pallas import tpu as pltpu
from jax.experimental.pallas import tpu_sc as plsc

N_NODES = 10000
NNZ = 320000
D_FEAT = 128

NUM_CORES = 2
NUM_SUBCORES = 16
NW = NUM_CORES * NUM_SUBCORES
PER_W = NNZ // NW
CHUNK = 80
N_CHUNKS = PER_W // CHUNK


@functools.partial(
    pl.kernel,
    out_type=jax.ShapeDtypeStruct((NNZ, D_FEAT), jnp.float32),
    mesh=plsc.VectorSubcoreMesh(core_axis_name="c", subcore_axis_name="s"),
    scratch_types=[
        pltpu.VMEM((N_CHUNKS, CHUNK), jnp.int32),
        pltpu.VMEM((CHUNK, D_FEAT), jnp.float32),
        pltpu.SemaphoreType.DMA,
    ],
)
def _unpool(x_hbm, idx_hbm, out_hbm, idx_v, rows_v, sem):
    wid = (lax.axis_index("s") * NUM_CORES + lax.axis_index("c")).astype(
        jnp.int32
    )
    base = wid * jnp.int32(PER_W)
    pltpu.sync_copy(idx_hbm.at[wid], idx_v)

    def body(c, _):
        pltpu.async_copy(x_hbm.at[idx_v.at[c]], rows_v, sem).wait()
        off = base + c * jnp.int32(CHUNK)
        pltpu.sync_copy(rows_v, out_hbm.at[pl.ds(off, CHUNK)])
        return ()

    lax.fori_loop(
        jnp.int32(0), jnp.int32(N_CHUNKS), body, (), unroll=False
    )


def kernel(X, tarX_indices):
    idx = tarX_indices[0].astype(jnp.int32).reshape(NW, N_CHUNKS, CHUNK)
    return _unpool(X, idx)


# speedup vs baseline: 2.8435x; 1.7478x over previous
import functools

import jax
import jax.numpy as jnp
from jax import lax
from jax.experimental import pallas as pl
from jax.experimental.pallas import tpu as pltpu
from jax.experimental.pallas import tpu_sc as plsc

N_NODES = 10000
NNZ = 320000
D_FEAT = 128

NUM_CORES = 2
NUM_SUBCORES = 16
NW = NUM_CORES * NUM_SUBCORES
PER_W = NNZ // NW
CHUNK = 80
N_CHUNKS = PER_W // CHUNK
NBUF = 5
N_GROUPS = N_CHUNKS // NBUF


@functools.partial(
    pl.kernel,
    out_type=jax.ShapeDtypeStruct((NNZ, D_FEAT), jnp.float32),
    mesh=plsc.VectorSubcoreMesh(core_axis_name="c", subcore_axis_name="s"),
    scratch_types=[
        pltpu.VMEM((N_CHUNKS, CHUNK), jnp.int32),
    ]
    + [pltpu.VMEM((CHUNK, D_FEAT), jnp.float32) for _ in range(NBUF)]
    + [pltpu.SemaphoreType.DMA for _ in range(NBUF)],
)
def _unpool(
    x_hbm, idx_hbm, out_hbm, idx_v,
    r0, r1, r2, r3, r4, s0, s1, s2, s3, s4,
):
    bufs = (r0, r1, r2, r3, r4)
    sems = (s0, s1, s2, s3, s4)
    wid = (lax.axis_index("s") * NUM_CORES + lax.axis_index("c")).astype(
        jnp.int32
    )
    base = wid * jnp.int32(PER_W)
    pltpu.sync_copy(idx_hbm.at[wid], idx_v)

    def start_gather(c, b):
        pltpu.async_copy(x_hbm.at[idx_v.at[c]], bufs[b], sems[b])

    def wait_buf_dma(b):
        pltpu.make_async_copy(
            out_hbm.at[pl.ds(jnp.int32(0), CHUNK)], bufs[b], sems[b]
        ).wait()

    def start_store(c, b):
        off = base + c * jnp.int32(CHUNK)
        pltpu.async_copy(bufs[b], out_hbm.at[pl.ds(off, CHUNK)], sems[b])

    for b in range(NBUF):
        start_gather(jnp.int32(b), b)
    for b in range(NBUF):
        wait_buf_dma(b)
        start_store(jnp.int32(b), b)

    def body(g, _):
        for b in range(NBUF):
            c = g * jnp.int32(NBUF) + jnp.int32(b)
            wait_buf_dma(b)
            start_gather(c, b)
        for b in range(NBUF):
            c = g * jnp.int32(NBUF) + jnp.int32(b)
            wait_buf_dma(b)
            start_store(c, b)
        return ()

    lax.fori_loop(jnp.int32(1), jnp.int32(N_GROUPS), body, (), unroll=False)

    for b in range(NBUF):
        wait_buf_dma(b)


def kernel(X, tarX_indices):
    idx = tarX_indices[0].astype(jnp.int32).reshape(NW, N_CHUNKS, CHUNK)
    return _unpool(X, idx)


# speedup vs baseline: 2.8791x; 1.0125x over previous
import functools

import jax
import jax.numpy as jnp
from jax import lax
from jax.experimental import pallas as pl
from jax.experimental.pallas import tpu as pltpu
from jax.experimental.pallas import tpu_sc as plsc

N_NODES = 10000
NNZ = 320000
D_FEAT = 128

NUM_CORES = 2
NUM_SUBCORES = 16
NW = NUM_CORES * NUM_SUBCORES
PER_W = NNZ // NW
CHUNK = 80
N_CHUNKS = PER_W // CHUNK
GPC = 5
GROUP = GPC * CHUNK
N_GROUPS = N_CHUNKS // GPC
N_PAIRS = N_GROUPS // 2


@functools.partial(
    pl.kernel,
    out_type=jax.ShapeDtypeStruct((NNZ, D_FEAT), jnp.float32),
    mesh=plsc.VectorSubcoreMesh(core_axis_name="c", subcore_axis_name="s"),
    scratch_types=[
        pltpu.VMEM((N_CHUNKS, CHUNK), jnp.int32),
        pltpu.VMEM((GROUP, D_FEAT), jnp.float32),
        pltpu.VMEM((GROUP, D_FEAT), jnp.float32),
        pltpu.SemaphoreType.DMA,
        pltpu.SemaphoreType.DMA,
    ],
)
def _unpool(x_hbm, idx_hbm, out_hbm, idx_v, stage_a, stage_b, sem_a, sem_b):
    stages = (stage_a, stage_b)
    sems = (sem_a, sem_b)
    wid = (lax.axis_index("s") * NUM_CORES + lax.axis_index("c")).astype(
        jnp.int32
    )
    base = wid * jnp.int32(PER_W)
    pltpu.sync_copy(idx_hbm.at[wid], idx_v)

    def start_gathers(g, s):
        for j in range(GPC):
            c = g * jnp.int32(GPC) + jnp.int32(j)
            pltpu.async_copy(
                x_hbm.at[idx_v.at[c]],
                stages[s].at[pl.ds(j * CHUNK, CHUNK)],
                sems[s],
            )

    def wait_group(s):
        pltpu.make_async_copy(
            out_hbm.at[pl.ds(jnp.int32(0), GROUP)], stages[s], sems[s]
        ).wait()

    def start_store(g, s):
        off = base + g * jnp.int32(GROUP)
        pltpu.async_copy(stages[s], out_hbm.at[pl.ds(off, GROUP)], sems[s])

    start_gathers(jnp.int32(0), 0)
    start_gathers(jnp.int32(1), 1)

    def body(p, _):
        ga = jnp.int32(2) * p
        gb = ga + jnp.int32(1)
        wait_group(0)
        start_store(ga, 0)
        wait_group(1)
        start_store(gb, 1)
        wait_group(0)
        start_gathers(ga + jnp.int32(2), 0)

        @pl.when(p < jnp.int32(N_PAIRS - 1))
        def _():
            wait_group(1)
            start_gathers(gb + jnp.int32(2), 1)

        return ()

    lax.fori_loop(jnp.int32(0), jnp.int32(N_PAIRS), body, (), unroll=False)

    wait_group(0)
    start_store(jnp.int32(N_GROUPS - 1), 0)
    wait_group(0)
    wait_group(1)


def kernel(X, tarX_indices):
    idx = tarX_indices[0].astype(jnp.int32).reshape(NW, N_CHUNKS, CHUNK)
    return _unpool(X, idx)


# speedup vs baseline: 8.2773x; 2.8750x over previous
import functools

import jax
import jax.numpy as jnp
from jax import lax
from jax.experimental import pallas as pl
from jax.experimental.pallas import tpu as pltpu
from jax.experimental.pallas import tpu_sc as plsc

N_NODES = 10000
NNZ = 320000
D_FEAT = 128

NUM_CORES = 2
NUM_SUBCORES = 16
NW = NUM_CORES * NUM_SUBCORES
PER_W = NNZ // NW
CHUNK = 80
N_CHUNKS = PER_W // CHUNK
NBUF = 3
N_ROUNDS = N_CHUNKS // NBUF - 1
TAIL_START = NBUF * (N_ROUNDS + 1)
TAIL = N_CHUNKS - TAIL_START


@functools.partial(
    pl.kernel,
    out_type=jax.ShapeDtypeStruct((NNZ, D_FEAT), jnp.float32),
    mesh=plsc.VectorSubcoreMesh(core_axis_name="c", subcore_axis_name="s"),
    scratch_types=[
        pltpu.VMEM_SHARED((N_NODES, D_FEAT), jnp.float32),
        pltpu.VMEM((N_CHUNKS, CHUNK), jnp.int32),
        pltpu.VMEM((CHUNK, D_FEAT), jnp.float32),
        pltpu.VMEM((CHUNK, D_FEAT), jnp.float32),
        pltpu.VMEM((CHUNK, D_FEAT), jnp.float32),
        pltpu.SemaphoreType.DMA,
        pltpu.SemaphoreType.DMA,
        pltpu.SemaphoreType.DMA,
    ],
)
def _unpool(
    x_hbm, idx_hbm, out_hbm, x_spmem, idx_v, r0, r1, r2, s0, s1, s2
):
    bufs = (r0, r1, r2)
    sems = (s0, s1, s2)
    sid = lax.axis_index("s").astype(jnp.int32)
    wid = sid * jnp.int32(NUM_CORES) + lax.axis_index("c").astype(jnp.int32)
    base = wid * jnp.int32(PER_W)

    @pl.when(sid == jnp.int32(0))
    def _():
        pltpu.sync_copy(x_hbm, x_spmem)

    pltpu.sync_copy(idx_hbm.at[wid], idx_v)
    plsc.subcore_barrier()

    def start_gather(c, b):
        pltpu.async_copy(x_spmem.at[idx_v.at[c]], bufs[b], sems[b])

    def wait_buf_dma(b):
        pltpu.make_async_copy(
            out_hbm.at[pl.ds(jnp.int32(0), CHUNK)], bufs[b], sems[b]
        ).wait()

    def start_store(c, b):
        off = base + c * jnp.int32(CHUNK)
        pltpu.async_copy(bufs[b], out_hbm.at[pl.ds(off, CHUNK)], sems[b])

    for b in range(NBUF):
        start_gather(jnp.int32(b), b)
    for b in range(NBUF):
        wait_buf_dma(b)
        start_store(jnp.int32(b), b)

    def body(g, _):
        for b in range(NBUF):
            c = g * jnp.int32(NBUF) + jnp.int32(b)
            wait_buf_dma(b)
            start_gather(c, b)
        for b in range(NBUF):
            c = g * jnp.int32(NBUF) + jnp.int32(b)
            wait_buf_dma(b)
            start_store(c, b)
        return ()

    lax.fori_loop(
        jnp.int32(1), jnp.int32(N_ROUNDS + 1), body, (), unroll=False
    )

    for t in range(TAIL):
        wait_buf_dma(t)
        start_gather(jnp.int32(TAIL_START + t), t)
    for t in range(TAIL):
        wait_buf_dma(t)
        start_store(jnp.int32(TAIL_START + t), t)

    for b in range(NBUF):
        wait_buf_dma(b)


def kernel(X, tarX_indices):
    idx = tarX_indices[0].astype(jnp.int32).reshape(NW, N_CHUNKS, CHUNK)
    return _unpool(X, idx)


# speedup vs baseline: 8.2965x; 1.0023x over previous
import functools

import jax
import jax.numpy as jnp
from jax import lax
from jax.experimental import pallas as pl
from jax.experimental.pallas import tpu as pltpu
from jax.experimental.pallas import tpu_sc as plsc

N_NODES = 10000
NNZ = 320000
D_FEAT = 128

NUM_CORES = 2
NUM_SUBCORES = 16
NW = NUM_CORES * NUM_SUBCORES
PER_W = NNZ // NW
CHUNK = 80
N_CHUNKS = PER_W // CHUNK
NBUF = 3
N_ROUNDS = N_CHUNKS // NBUF - 1
TAIL_START = NBUF * (N_ROUNDS + 1)
TAIL = N_CHUNKS - TAIL_START
STAGE_ROWS = 624
STAGE_LAST = N_NODES - 15 * STAGE_ROWS


@functools.partial(
    pl.kernel,
    out_type=jax.ShapeDtypeStruct((NNZ, D_FEAT), jnp.float32),
    mesh=plsc.VectorSubcoreMesh(core_axis_name="c", subcore_axis_name="s"),
    scratch_types=[
        pltpu.VMEM_SHARED((N_NODES, D_FEAT), jnp.float32),
        pltpu.VMEM((N_CHUNKS, CHUNK), jnp.int32),
        pltpu.VMEM((CHUNK, D_FEAT), jnp.float32),
        pltpu.VMEM((CHUNK, D_FEAT), jnp.float32),
        pltpu.VMEM((CHUNK, D_FEAT), jnp.float32),
        pltpu.SemaphoreType.DMA,
        pltpu.SemaphoreType.DMA,
        pltpu.SemaphoreType.DMA,
    ],
)
def _unpool(
    x_hbm, idx_hbm, out_hbm, x_spmem, idx_v, r0, r1, r2, s0, s1, s2
):
    bufs = (r0, r1, r2)
    sems = (s0, s1, s2)
    sid = lax.axis_index("s").astype(jnp.int32)
    wid = sid * jnp.int32(NUM_CORES) + lax.axis_index("c").astype(jnp.int32)
    base = wid * jnp.int32(PER_W)

    xoff = sid * jnp.int32(STAGE_ROWS)

    @pl.when(sid < jnp.int32(NUM_SUBCORES - 1))
    def _():
        pltpu.sync_copy(
            x_hbm.at[pl.ds(xoff, STAGE_ROWS)],
            x_spmem.at[pl.ds(xoff, STAGE_ROWS)],
        )

    @pl.when(sid == jnp.int32(NUM_SUBCORES - 1))
    def _():
        pltpu.sync_copy(
            x_hbm.at[pl.ds(xoff, STAGE_LAST)],
            x_spmem.at[pl.ds(xoff, STAGE_LAST)],
        )

    pltpu.sync_copy(idx_hbm.at[wid], idx_v)
    plsc.subcore_barrier()

    def start_gather(c, b):
        pltpu.async_copy(x_spmem.at[idx_v.at[c]], bufs[b], sems[b])

    def wait_buf_dma(b):
        pltpu.make_async_copy(
            out_hbm.at[pl.ds(jnp.int32(0), CHUNK)], bufs[b], sems[b]
        ).wait()

    def start_store(c, b):
        off = base + c * jnp.int32(CHUNK)
        pltpu.async_copy(bufs[b], out_hbm.at[pl.ds(off, CHUNK)], sems[b])

    for b in range(NBUF):
        start_gather(jnp.int32(b), b)
    for b in range(NBUF):
        wait_buf_dma(b)
        start_store(jnp.int32(b), b)

    def body(g, _):
        for b in range(NBUF):
            c = g * jnp.int32(NBUF) + jnp.int32(b)
            wait_buf_dma(b)
            start_gather(c, b)
        for b in range(NBUF):
            c = g * jnp.int32(NBUF) + jnp.int32(b)
            wait_buf_dma(b)
            start_store(c, b)
        return ()

    lax.fori_loop(
        jnp.int32(1), jnp.int32(N_ROUNDS + 1), body, (), unroll=False
    )

    for t in range(TAIL):
        wait_buf_dma(t)
        start_gather(jnp.int32(TAIL_START + t), t)
    for t in range(TAIL):
        wait_buf_dma(t)
        start_store(jnp.int32(TAIL_START + t), t)

    for b in range(NBUF):
        wait_buf_dma(b)


def kernel(X, tarX_indices):
    idx = tarX_indices[0].astype(jnp.int32).reshape(NW, N_CHUNKS, CHUNK)
    return _unpool(X, idx)
